# Initial kernel scaffold; baseline (speedup 1.0000x reference)
import jax
import jax.numpy as jnp
from jax.experimental import pallas as pl


def kernel(userIdx, itemIdx, edge_index, edge_weight, emb_user, emb_item, W1, b1, W2, b2, W3, b3):
    raise NotImplementedError("write your pallas kernel here")


# speedup vs baseline: 1.0006x; 1.0006x over previous
import functools
import jax
import jax.numpy as jnp
from jax.experimental import pallas as pl

USER_NUM = 20000
N_NODES = 50000
NUM_LAYERS = 3


def _mlp_body(u_ref, i_ref, w1u_ref, w1i_ref, b1_ref, w2_ref, b2_ref, w3_ref, b3_ref, o_ref):
    h = jnp.dot(u_ref[...], w1u_ref[...], preferred_element_type=jnp.float32)
    h += jnp.dot(i_ref[...], w1i_ref[...], preferred_element_type=jnp.float32)
    h = jax.nn.relu(h + b1_ref[...])
    h2 = jnp.dot(h, w2_ref[...], preferred_element_type=jnp.float32) + b2_ref[...]
    o_ref[...] = jnp.dot(h2, w3_ref[...], preferred_element_type=jnp.float32) + b3_ref[...]


def _mlp(u, i, W1, b1, W2, b2, W3, b3):
    B = u.shape[0]
    out = pl.pallas_call(
        _mlp_body,
        out_shape=jax.ShapeDtypeStruct((B, 1), jnp.float32),
    )(u, i, W1[:64], W1[64:], b1[None, :], W2, b2[None, :], W3, b3[None, :])
    return out.reshape(-1)


def kernel(userIdx, itemIdx, edge_index, edge_weight, emb_user, emb_item, W1, b1, W2, b2, W3, b3):
    all_embs = jnp.concatenate([emb_user, emb_item], axis=0)
    dst = edge_index[0]
    src = edge_index[1]
    x = all_embs
    final = all_embs
    for k in range(NUM_LAYERS):
        msg = edge_weight[:, None] * jnp.take(x, src, axis=0)
        x = jax.ops.segment_sum(msg, dst, num_segments=N_NODES)
        x = x * (1.0 / (k + 2))
        final = final + x
    u = jnp.take(final, userIdx, axis=0)
    i = jnp.take(final, itemIdx + USER_NUM, axis=0)
    return _mlp(u, i, W1, b1, W2, b2, W3, b3)


# speedup vs baseline: 2.4840x; 2.4825x over previous
import functools

import jax
import jax.numpy as jnp
from jax import lax
from jax.experimental import pallas as pl
from jax.experimental.pallas import tpu as pltpu
from jax.experimental.pallas import tpu_sc as plsc

USER_NUM = 20000
N_NODES = 50000
EMBED = 64
NUM_LAYERS = 3
BATCH = 4096

NP = 50176
HALF = NP // 2
TRASH = 16
ACC_ROWS = HALF + TRASH
Z_SLICE = ACC_ROWS // 16
E_PAD = 802816
CHUNK = 256
GRP = CHUNK // 128
EDGE_ROWS = E_PAD // 128
ROWS_PER_SUB = EDGE_ROWS // 16
NCHUNKS = ROWS_PER_SUB // GRP
ZROWS = 32


_LANE_DNUMS = lax.GatherDimensionNumbers(
    offset_dims=(), collapsed_slice_dims=(0,), start_index_map=(0,))


def _lane_bcast(vec, lane):
    idx = jnp.full((16, 1), lane, jnp.int32)
    return lax.gather(vec, idx, _LANE_DNUMS, (1,),
                      mode=lax.GatherScatterMode.PROMISE_IN_BOUNDS)


def _layer_body(scale, x_hbm, src_hbm, dst_hbm, w_hbm, out_hbm,
                srcv, dstv, dlv, wv, rows, zbuf, acc, sem):
    c = lax.axis_index("c")
    s = lax.axis_index("s")

    def zz(e, _):
        z = jnp.zeros((16,), jnp.float32)
        for j in range(EMBED // 16):
            zbuf[e, pl.ds(16 * j, 16)] = z
        return 0
    lax.fori_loop(0, ZROWS, zz, 0)
    for i in range(Z_SLICE // ZROWS):
        pltpu.sync_copy(zbuf, acc.at[pl.ds(s * Z_SLICE + i * ZROWS, ZROWS)])
    rem = Z_SLICE % ZROWS
    if rem:
        pltpu.sync_copy(zbuf.at[pl.ds(0, rem)],
                        acc.at[pl.ds(s * Z_SLICE + (Z_SLICE // ZROWS) * ZROWS, rem)])
    plsc.subcore_barrier()

    half_i = jnp.full((16,), HALF, jnp.int32)
    chalf = (c * HALF).astype(jnp.int32)

    def chunk_body(t, _):
        r0 = s * ROWS_PER_SUB + t * GRP
        pltpu.sync_copy(src_hbm.at[pl.ds(r0, GRP)], srcv)
        pltpu.sync_copy(dst_hbm.at[pl.ds(r0, GRP)], dstv)
        pltpu.sync_copy(w_hbm.at[pl.ds(r0 * 128, CHUNK)], wv)
        cps = [pltpu.async_copy(x_hbm.at[srcv.at[g]], rows.at[g], sem)
               for g in range(GRP)]
        for g in range(GRP):
            for k in range(8):
                d = dstv[g, pl.ds(16 * k, 16)] - chalf
                ok = (d >= 0) & (d < half_i)
                dlv[g, pl.ds(16 * k, 16)] = jnp.where(ok, d, half_i)
        for cp in cps:
            cp.wait()

        for g in range(GRP):
            def sc_body(b, _):
                wvec = wv[pl.ds(g * 128 + 16 * b, 16)] * scale
                for l in range(16):
                    wb = _lane_bcast(wvec, l)
                    e = 16 * b + l
                    for j in range(EMBED // 16):
                        rows[g, e, pl.ds(16 * j, 16)] = rows[g, e, pl.ds(16 * j, 16)] * wb
                return 0
            lax.fori_loop(0, 8, sc_body, 0)

        for g in range(GRP):
            pltpu.sync_copy(rows.at[g], acc.at[dlv.at[g]], add=True)
        return 0

    lax.fori_loop(0, NCHUNKS, chunk_body, 0)
    plsc.subcore_barrier()

    wb_rows = HALF // 16
    pltpu.sync_copy(acc.at[pl.ds(s * wb_rows, wb_rows)],
                    out_hbm.at[pl.ds(c * HALF + s * wb_rows, wb_rows)])


@functools.lru_cache(maxsize=None)
def _make_layer(scale):
    mesh = plsc.VectorSubcoreMesh(core_axis_name="c", subcore_axis_name="s")
    return pl.kernel(
        functools.partial(_layer_body, scale),
        out_type=jax.ShapeDtypeStruct((NP, EMBED), jnp.float32),
        mesh=mesh,
        scratch_types=[
            pltpu.VMEM((GRP, 128), jnp.int32),
            pltpu.VMEM((GRP, 128), jnp.int32),
            pltpu.VMEM((GRP, 128), jnp.int32),
            pltpu.VMEM((CHUNK,), jnp.float32),
            pltpu.VMEM((GRP, 128, EMBED), jnp.float32),
            pltpu.VMEM((ZROWS, EMBED), jnp.float32),
            pltpu.VMEM_SHARED((ACC_ROWS, EMBED), jnp.float32),
            pltpu.SemaphoreType.DMA,
        ],
        compiler_params=pltpu.CompilerParams(use_tc_tiling_on_sc=False),
        name=f"lgcn_layer_{int(1.0/scale)}",
    )


def _final_body(x0, x1, x2, x3, uidx_hbm, iidx_hbm, u_hbm, i_hbm,
                idxv, g0, g1, g2, g3, sem):
    c = lax.axis_index("c")
    s = lax.axis_index("s")
    wid = s * 2 + c
    base = wid * (BATCH // 32)

    def do(idx_hbm, off, out_hbm):
        pltpu.sync_copy(idx_hbm.at[pl.ds(base, BATCH // 32)], idxv)
        if off:
            offv = jnp.full((16,), off, jnp.int32)
            for k in range(BATCH // 32 // 16):
                idxv[pl.ds(16 * k, 16)] = idxv[pl.ds(16 * k, 16)] + offv
        cps = [pltpu.async_copy(x.at[idxv], g, sem)
               for x, g in ((x0, g0), (x1, g1), (x2, g2), (x3, g3))]
        for cp in cps:
            cp.wait()

        def sum_body(e, _):
            for j in range(EMBED // 16):
                d = pl.ds(16 * j, 16)
                g0[e, d] = g0[e, d] + g1[e, d] + g2[e, d] + g3[e, d]
            return 0
        lax.fori_loop(0, BATCH // 32, sum_body, 0)
        pltpu.sync_copy(g0, out_hbm.at[pl.ds(base, BATCH // 32)])

    do(uidx_hbm, 0, u_hbm)
    do(iidx_hbm, USER_NUM, i_hbm)


@functools.lru_cache(maxsize=None)
def _make_final():
    mesh = plsc.VectorSubcoreMesh(core_axis_name="c", subcore_axis_name="s")
    return pl.kernel(
        _final_body,
        out_type=(jax.ShapeDtypeStruct((BATCH, EMBED), jnp.float32),
                  jax.ShapeDtypeStruct((BATCH, EMBED), jnp.float32)),
        mesh=mesh,
        scratch_types=[
            pltpu.VMEM((BATCH // 32,), jnp.int32),
            pltpu.VMEM((BATCH // 32, EMBED), jnp.float32),
            pltpu.VMEM((BATCH // 32, EMBED), jnp.float32),
            pltpu.VMEM((BATCH // 32, EMBED), jnp.float32),
            pltpu.VMEM((BATCH // 32, EMBED), jnp.float32),
            pltpu.SemaphoreType.DMA,
        ],
        compiler_params=pltpu.CompilerParams(use_tc_tiling_on_sc=False),
        name="lgcn_final_gather",
    )


def _mlp_body(u_ref, i_ref, w1u_ref, w1i_ref, b1_ref, w2_ref, b2_ref, w3_ref, b3_ref, o_ref):
    h = jnp.dot(u_ref[...], w1u_ref[...], preferred_element_type=jnp.float32)
    h += jnp.dot(i_ref[...], w1i_ref[...], preferred_element_type=jnp.float32)
    h = jax.nn.relu(h + b1_ref[...])
    h2 = jnp.dot(h, w2_ref[...], preferred_element_type=jnp.float32) + b2_ref[...]
    o_ref[...] = jnp.dot(h2, w3_ref[...], preferred_element_type=jnp.float32) + b3_ref[...]


def _mlp(u, i, W1, b1, W2, b2, W3, b3):
    out = pl.pallas_call(
        _mlp_body,
        out_shape=jax.ShapeDtypeStruct((BATCH, 1), jnp.float32),
    )(u, i, W1[:EMBED], W1[EMBED:], b1[None, :], W2, b2[None, :], W3, b3[None, :])
    return out.reshape(-1)


def kernel(userIdx, itemIdx, edge_index, edge_weight, emb_user, emb_item, W1, b1, W2, b2, W3, b3):
    n_edges = edge_weight.shape[0]
    x0 = jnp.zeros((NP, EMBED), jnp.float32)
    x0 = x0.at[:USER_NUM].set(emb_user).at[USER_NUM:N_NODES].set(emb_item)
    dst = jnp.zeros((E_PAD,), jnp.int32).at[:n_edges].set(edge_index[0]).reshape(EDGE_ROWS, 128)
    src = jnp.zeros((E_PAD,), jnp.int32).at[:n_edges].set(edge_index[1]).reshape(EDGE_ROWS, 128)
    w = jnp.zeros((E_PAD,), jnp.float32).at[:n_edges].set(edge_weight)

    x1 = _make_layer(1.0 / 2)(x0, src, dst, w)
    x2 = _make_layer(1.0 / 3)(x1, src, dst, w)
    x3 = _make_layer(1.0 / 4)(x2, src, dst, w)

    u, i = _make_final()(x0, x1, x2, x3, userIdx, itemIdx)
    return _mlp(u, i, W1, b1, W2, b2, W3, b3)


# speedup vs baseline: 5.6895x; 2.2904x over previous
import functools

import jax
import jax.numpy as jnp
from jax import lax
from jax.experimental import pallas as pl
from jax.experimental.pallas import tpu as pltpu
from jax.experimental.pallas import tpu_sc as plsc

USER_NUM = 20000
N_NODES = 50000
EMBED = 64
NUM_LAYERS = 3
BATCH = 4096

NP = 50176
HALF = NP // 2
TRASH = 16
ACC_ROWS = HALF + TRASH
Z_SLICE = ACC_ROWS // 16
E_PAD = 802816
CHUNK = 128
EDGE_ROWS = E_PAD // 128
ROWS_PER_SUB = EDGE_ROWS // 16


_LANE_DNUMS = lax.GatherDimensionNumbers(
    offset_dims=(), collapsed_slice_dims=(0,), start_index_map=(0,))


def _lane_bcast(vec, lane):
    idx = jnp.full((16, 1), lane, jnp.int32)
    return lax.gather(vec, idx, _LANE_DNUMS, (1,),
                      mode=lax.GatherScatterMode.PROMISE_IN_BOUNDS)


def _layer_body(scale, x_hbm, edata_hbm, out_hbm,
                ev0, ev1, dlv, rows0, rows1, acc, sem_i, sem_g0, sem_g1):
    c = lax.axis_index("c")
    s = lax.axis_index("s")
    base = s * ROWS_PER_SUB

    def zz(e, _):
        z = jnp.zeros((16,), jnp.float32)
        for j in range(EMBED // 16):
            rows0[e, pl.ds(16 * j, 16)] = z
        return 0
    lax.fori_loop(0, CHUNK, zz, 0)
    for i in range(Z_SLICE // CHUNK):
        pltpu.sync_copy(rows0, acc.at[pl.ds(s * Z_SLICE + i * CHUNK, CHUNK)])
    rem = Z_SLICE % CHUNK
    if rem:
        pltpu.sync_copy(rows0.at[pl.ds(0, rem)],
                        acc.at[pl.ds(s * Z_SLICE + (Z_SLICE // CHUNK) * CHUNK, rem)])

    half_i = jnp.full((16,), HALF, jnp.int32)
    chalf = (c * HALF).astype(jnp.int32)

    pltpu.async_copy(edata_hbm.at[base], ev0, sem_i).wait()
    pltpu.async_copy(x_hbm.at[ev0.at[0]], rows0, sem_g0)
    pltpu.async_copy(edata_hbm.at[base + 1], ev1, sem_i)
    plsc.subcore_barrier()

    bufs = ((ev0, rows0, sem_g0), (ev1, rows1, sem_g1))

    def step(t, cur, nxt):
        ebuf, rows, sem_g = cur
        ebuf_n, rows_n, sem_g_n = nxt
        pltpu.make_async_copy(edata_hbm.at[base + t + 1], ebuf_n, sem_i).wait()
        pltpu.async_copy(x_hbm.at[ebuf_n.at[0]], rows_n, sem_g_n)
        for k in range(8):
            d = ebuf[1, pl.ds(16 * k, 16)] - chalf
            ok = (d >= 0) & (d < half_i)
            dlv[0, pl.ds(16 * k, 16)] = jnp.where(ok, d, half_i)
        wvecs = [plsc.bitcast(ebuf[2, pl.ds(16 * b, 16)], jnp.float32) * scale
                 for b in range(8)]
        pltpu.make_async_copy(x_hbm.at[ebuf.at[0]], rows, sem_g).wait()
        pltpu.async_copy(edata_hbm.at[base + t + 2], ebuf, sem_i)
        for b in range(8):
            for l in range(16):
                wb = _lane_bcast(wvecs[b], l)
                e = 16 * b + l
                for j in range(EMBED // 16):
                    rows[e, pl.ds(16 * j, 16)] = rows[e, pl.ds(16 * j, 16)] * wb
        pltpu.sync_copy(rows, acc.at[dlv.at[0]], add=True)

    def pair_body(i, _):
        t = i * 2
        step(t, bufs[0], bufs[1])
        step(t + 1, bufs[1], bufs[0])
        return 0
    lax.fori_loop(0, ROWS_PER_SUB // 2, pair_body, 0)

    pltpu.make_async_copy(x_hbm.at[ev0.at[0]], rows0, sem_g0).wait()
    pltpu.make_async_copy(edata_hbm.at[base], ev1, sem_i).wait()
    plsc.subcore_barrier()

    wb_rows = HALF // 16
    pltpu.sync_copy(acc.at[pl.ds(s * wb_rows, wb_rows)],
                    out_hbm.at[pl.ds(c * HALF + s * wb_rows, wb_rows)])


@functools.lru_cache(maxsize=None)
def _make_layer(scale):
    mesh = plsc.VectorSubcoreMesh(core_axis_name="c", subcore_axis_name="s")
    return pl.kernel(
        functools.partial(_layer_body, scale),
        out_type=jax.ShapeDtypeStruct((NP, EMBED), jnp.float32),
        mesh=mesh,
        scratch_types=[
            pltpu.VMEM((3, 128), jnp.int32),
            pltpu.VMEM((3, 128), jnp.int32),
            pltpu.VMEM((1, 128), jnp.int32),
            pltpu.VMEM((CHUNK, EMBED), jnp.float32),
            pltpu.VMEM((CHUNK, EMBED), jnp.float32),
            pltpu.VMEM_SHARED((ACC_ROWS, EMBED), jnp.float32),
            pltpu.SemaphoreType.DMA,
            pltpu.SemaphoreType.DMA,
            pltpu.SemaphoreType.DMA,
        ],
        compiler_params=pltpu.CompilerParams(use_tc_tiling_on_sc=False, needs_layout_passes=False),
        name=f"lgcn_layer_{int(1.0/scale)}",
    )


def _final_body(x0, x1, x2, x3, uidx_hbm, iidx_hbm, u_hbm, i_hbm,
                idxv, g0, g1, g2, g3, sem):
    c = lax.axis_index("c")
    s = lax.axis_index("s")
    wid = s * 2 + c
    base = wid * (BATCH // 32)

    def do(idx_hbm, off, out_hbm):
        pltpu.sync_copy(idx_hbm.at[pl.ds(base, BATCH // 32)], idxv)
        if off:
            offv = jnp.full((16,), off, jnp.int32)
            for k in range(BATCH // 32 // 16):
                idxv[pl.ds(16 * k, 16)] = idxv[pl.ds(16 * k, 16)] + offv
        cps = [pltpu.async_copy(x.at[idxv], g, sem)
               for x, g in ((x0, g0), (x1, g1), (x2, g2), (x3, g3))]
        for cp in cps:
            cp.wait()

        def sum_body(e, _):
            for j in range(EMBED // 16):
                d = pl.ds(16 * j, 16)
                g0[e, d] = g0[e, d] + g1[e, d] + g2[e, d] + g3[e, d]
            return 0
        lax.fori_loop(0, BATCH // 32, sum_body, 0)
        pltpu.sync_copy(g0, out_hbm.at[pl.ds(base, BATCH // 32)])

    do(uidx_hbm, 0, u_hbm)
    do(iidx_hbm, USER_NUM, i_hbm)


@functools.lru_cache(maxsize=None)
def _make_final():
    mesh = plsc.VectorSubcoreMesh(core_axis_name="c", subcore_axis_name="s")
    return pl.kernel(
        _final_body,
        out_type=(jax.ShapeDtypeStruct((BATCH, EMBED), jnp.float32),
                  jax.ShapeDtypeStruct((BATCH, EMBED), jnp.float32)),
        mesh=mesh,
        scratch_types=[
            pltpu.VMEM((BATCH // 32,), jnp.int32),
            pltpu.VMEM((BATCH // 32, EMBED), jnp.float32),
            pltpu.VMEM((BATCH // 32, EMBED), jnp.float32),
            pltpu.VMEM((BATCH // 32, EMBED), jnp.float32),
            pltpu.VMEM((BATCH // 32, EMBED), jnp.float32),
            pltpu.SemaphoreType.DMA,
        ],
        compiler_params=pltpu.CompilerParams(use_tc_tiling_on_sc=False, needs_layout_passes=False),
        name="lgcn_final_gather",
    )


def _mlp_body(u_ref, i_ref, w1u_ref, w1i_ref, b1_ref, w2_ref, b2_ref, w3_ref, b3_ref, o_ref):
    h = jnp.dot(u_ref[...], w1u_ref[...], preferred_element_type=jnp.float32)
    h += jnp.dot(i_ref[...], w1i_ref[...], preferred_element_type=jnp.float32)
    h = jax.nn.relu(h + b1_ref[...])
    h2 = jnp.dot(h, w2_ref[...], preferred_element_type=jnp.float32) + b2_ref[...]
    o_ref[...] = jnp.dot(h2, w3_ref[...], preferred_element_type=jnp.float32) + b3_ref[...]


def _mlp(u, i, W1, b1, W2, b2, W3, b3):
    out = pl.pallas_call(
        _mlp_body,
        out_shape=jax.ShapeDtypeStruct((BATCH, 1), jnp.float32),
    )(u, i, W1[:EMBED], W1[EMBED:], b1[None, :], W2, b2[None, :], W3, b3[None, :])
    return out.reshape(-1)


def kernel(userIdx, itemIdx, edge_index, edge_weight, emb_user, emb_item, W1, b1, W2, b2, W3, b3):
    n_edges = edge_weight.shape[0]
    x0 = jnp.zeros((NP, EMBED), jnp.float32)
    x0 = x0.at[:USER_NUM].set(emb_user).at[USER_NUM:N_NODES].set(emb_item)
    dst = jnp.zeros((E_PAD,), jnp.int32).at[:n_edges].set(edge_index[0]).reshape(EDGE_ROWS, 128)
    src = jnp.zeros((E_PAD,), jnp.int32).at[:n_edges].set(edge_index[1]).reshape(EDGE_ROWS, 128)
    wbits = jax.lax.bitcast_convert_type(
        jnp.zeros((E_PAD,), jnp.float32).at[:n_edges].set(edge_weight),
        jnp.int32).reshape(EDGE_ROWS, 128)
    edata = jnp.zeros((EDGE_ROWS + 2, 3, 128), jnp.int32)
    edata = edata.at[:EDGE_ROWS].set(jnp.stack([src, dst, wbits], axis=1))

    x1 = _make_layer(1.0 / 2)(x0, edata)
    x2 = _make_layer(1.0 / 3)(x1, edata)
    x3 = _make_layer(1.0 / 4)(x2, edata)

    u, i = _make_final()(x0, x1, x2, x3, userIdx, itemIdx)
    return _mlp(u, i, W1, b1, W2, b2, W3, b3)
